# baseline (speedup 1.0000x reference)
import functools

import jax
import jax.numpy as jnp
from jax import lax
from jax.experimental import pallas as pl
from jax.experimental.pallas import tpu as pltpu
from jax.experimental.pallas import tpu_sc as plsc

D = 128
B = 1024
S = 200
L = 16
NC, NS = 2, 16
NW = NC * NS
ROWS_PER_W = B // NW
G0, G1 = 96, 104
FLAT = B * S


@jax.jit
def _sc_embed(x_flat, token_table, pos_table):
    mesh = plsc.VectorSubcoreMesh(core_axis_name="c", subcore_axis_name="s")

    @functools.partial(
        pl.kernel,
        mesh=mesh,
        out_type=jax.ShapeDtypeStruct((FLAT, D), jnp.float32),
        scratch_types=[
            pltpu.VMEM((S * ROWS_PER_W,), jnp.int32),
            pltpu.VMEM((S, D), jnp.float32),
            pltpu.VMEM_SHARED((NS, 2, G1, D), jnp.float32),
            pltpu.VMEM((S, D), jnp.float32),
            pltpu.VMEM((S, D), jnp.float32),
            pltpu.SemaphoreType.DMA,
            pltpu.SemaphoreType.DMA,
            pltpu.SemaphoreType.DMA,
            pltpu.SemaphoreType.DMA,
            pltpu.SemaphoreType.DMA,
            pltpu.SemaphoreType.DMA,
            pltpu.SemaphoreType.DMA,
            pltpu.SemaphoreType.DMA,
            pltpu.SemaphoreType.DMA,
        ],
    )
    def k(tok_hbm, pos_hbm, idx_hbm, out_hbm, idx_v, pos_v, shared,
          buf0, buf1, gsem0, gsem1, s1sem0, s1sem1, odsem0, odsem1,
          s2sem0, s2sem1, psem):
        bufs = (buf0, buf1)
        gsem = (gsem0, gsem1)
        s1sem = (s1sem0, s1sem1)
        odsem = (odsem0, odsem1)
        s2sem = (s2sem0, s2sem1)

        sid = lax.axis_index("s")
        wid = sid * NC + lax.axis_index("c")
        wbase = wid * (S * ROWS_PER_W)
        pltpu.sync_copy(idx_hbm.at[pl.ds(wbase, S * ROWS_PER_W)], idx_v)
        pos_copy = pltpu.async_copy(pos_hbm, pos_v, psem)

        def fire_gather(r, kb):
            pltpu.async_copy(
                tok_hbm.at[idx_v.at[pl.ds(r * S, G0)]],
                bufs[kb].at[pl.ds(0, G0)], gsem[kb])
            pltpu.async_copy(
                tok_hbm.at[idx_v.at[pl.ds(r * S + G0, G1)]],
                bufs[kb].at[pl.ds(G0, G1)], gsem[kb])

        def drain_gather(kb):
            pltpu.make_async_copy(
                tok_hbm.at[pl.ds(0, S)], bufs[kb], gsem[kb]).wait()

        def fire_split_out(r, kb):
            pltpu.async_copy(
                bufs[kb].at[pl.ds(0, G0)],
                out_hbm.at[pl.ds(wbase + r * S, G0)], odsem[kb])
            pltpu.async_copy(
                bufs[kb].at[pl.ds(G0, G1)], shared.at[sid, kb], s1sem[kb])

        def drain_split_out(kb):
            pltpu.make_async_copy(
                bufs[kb].at[pl.ds(0, G0)],
                out_hbm.at[pl.ds(0, G0)], odsem[kb]).wait()
            pltpu.make_async_copy(
                bufs[kb].at[pl.ds(G0, G1)], shared.at[sid, kb],
                s1sem[kb]).wait()

        def fire_s2(r, kb):
            pltpu.async_copy(
                shared.at[sid, kb],
                out_hbm.at[pl.ds(wbase + r * S + G0, G1)], s2sem[kb])

        def drain_s2(kb):
            pltpu.make_async_copy(
                shared.at[sid, kb], out_hbm.at[pl.ds(0, G1)],
                s2sem[kb]).wait()

        def add_pos(kb):
            buf = bufs[kb]

            @pl.loop(0, S)
            def _(i):
                for c in range(0, D, L):
                    plsc.addupdate(buf.at[i, pl.ds(c, L)],
                                   pos_v[i, pl.ds(c, L)])

        fire_gather(0, 0)
        pos_copy.wait()

        @pl.loop(0, ROWS_PER_W + 2, step=2)
        def _(r0):
            for kb in range(2):
                r = r0 + kb
                kp = 1 - kb
                cond = (r >= 1) & (r < ROWS_PER_W + 1)

                @pl.when(cond)
                def _():
                    drain_split_out(kp)
                    fire_s2(r - 1, kp)

                @pl.when(r + 1 < ROWS_PER_W)
                def _():
                    fire_gather(r + 1, kp)

                @pl.when(r < ROWS_PER_W)
                def _():
                    drain_gather(kb)
                    add_pos(kb)

                    @pl.when(r >= 2)
                    def _():
                        drain_s2(kb)
                    fire_split_out(r, kb)

        drain_s2(0)
        drain_s2(1)

    return k(token_table, pos_table, x_flat)


def kernel(x, token_table, pos_table):
    x_flat = x.reshape(FLAT).astype(jnp.int32)
    out = _sc_embed(x_flat, token_table, pos_table)
    return out.reshape(B, S, D)
